# baseline (speedup 1.0000x reference)
import functools

import jax
import jax.numpy as jnp
from jax import lax
from jax.experimental import pallas as pl
from jax.experimental.pallas import tpu as pltpu
from jax.experimental.pallas import tpu_sc as plsc

N = 10000
D = 128
G = 8
NP = 10240
NDEN = NP // D
NP_TOT = NP + NDEN
NC, NS = 2, 16
C = 64
E_RAW = 320000
E_TOT = E_RAW + N
NCHUNK = -(-E_TOT // (NC * NS * C))
T = NCHUNK * C
E_PAD = NC * NS * T
E_ALL = E_PAD + C

ROWS_PER_TILE = NP // NS
ZROWS = C


def _lane_take(v, idx):
    return lax.gather(
        v, idx[:, None],
        dimension_numbers=lax.GatherDimensionNumbers(
            offset_dims=(), collapsed_slice_dims=(0,), start_index_map=(0,)),
        slice_sizes=(1,), mode=lax.GatherScatterMode.PROMISE_IN_BOUNDS)


def _lane_allsum(v, i16):
    for b in (8, 4, 2, 1):
        v = v + _lane_take(v, i16 ^ b)
    return v


def _edge_kernel_body(xl_hbm, xr_hbm, src_hbm, dst_hbm, att_hbm, out_hbm,
                      acc, idx_s0, idx_d0, idx_s1, idx_d1, idx_den,
                      rows_s0, rows_d0, rows_s1, rows_d1,
                      oh_buf, att_v, sem0, sem1):
    c = lax.axis_index("c")
    s = lax.axis_index("s")
    tid = c * NS + s

    zv = jnp.zeros((16,), jnp.float32)

    def zb_body(r, _):
        for j in range(D // 16):
            rows_s0[r, pl.ds(j * 16, 16)] = zv
            oh_buf[r, pl.ds(j * 16, 16)] = zv
        return 0
    lax.fori_loop(0, ZROWS, zb_body, 0)
    row0 = s * ROWS_PER_TILE
    for b in range(ROWS_PER_TILE // ZROWS):
        pltpu.sync_copy(rows_s0, acc.at[pl.ds(row0 + b * ZROWS, ZROWS)])

    @pl.when(s == 0)
    def _():
        pltpu.sync_copy(rows_s0, acc.at[pl.ds(NP, ZROWS)])
        pltpu.sync_copy(rows_s0.at[pl.ds(0, NDEN - ZROWS)],
                        acc.at[pl.ds(NP + ZROWS, NDEN - ZROWS)])

    pltpu.sync_copy(att_hbm, att_v)
    att_regs = [att_v[pl.ds(k * 16, 16)] for k in range(D // 16)]
    i16 = lax.iota(jnp.int32, 16)
    col_iotas = [i16 + k * 16 for k in range(D // 16)]

    plsc.subcore_barrier()

    ebase = tid * T

    def stage_idx(g, idx_s, idx_d):
        eb = ebase + g * C
        pltpu.sync_copy(src_hbm.at[pl.ds(eb, C)], idx_s)
        pltpu.sync_copy(dst_hbm.at[pl.ds(eb, C)], idx_d)

    def issue(idx_s, idx_d, rs, rd, sem):
        pltpu.async_copy(xl_hbm.at[idx_s], rs, sem)
        pltpu.async_copy(xr_hbm.at[idx_d], rd, sem)

    def drain(idx_s, idx_d, rs, rd, sem):
        pltpu.make_async_copy(xl_hbm.at[idx_s], rs, sem).wait()
        pltpu.make_async_copy(xr_hbm.at[idx_d], rd, sem).wait()

    def compute_scatter(idx_d, rows_s, rows_d):
        @plsc.parallel_loop(0, C // 16, 1)
        def gbody(q):
            dvec = idx_d[pl.ds(q * 16, 16)]
            idx_den[pl.ds(q * 16, 16)] = (dvec >> 7) + NP
            colb = dvec & 127
            for l in range(16):
                e = q * 16 + l
                svecs = []
                accv = jnp.zeros((16,), jnp.float32)
                for k in range(D // 16):
                    a = rows_s[e, pl.ds(k * 16, 16)]
                    svecs.append(a)
                    t = a + rows_d[e, pl.ds(k * 16, 16)]
                    lr = jnp.maximum(t, 0.2 * t)
                    accv = accv + lr * att_regs[k]
                wv = jnp.exp(_lane_allsum(accv, i16))
                col = colb[l]
                mv = lax.broadcast(col & 15, (16,))
                oh_buf[e, pl.ds((col >> 4) * 16, 16)] = jnp.where(
                    i16 == mv, wv, 0.0)
                for k in range(D // 16):
                    rows_s[e, pl.ds(k * 16, 16)] = svecs[k] * wv

        pltpu.sync_copy(rows_s, acc.at[idx_d], add=True)
        pltpu.sync_copy(oh_buf, acc.at[idx_den], add=True)

        @plsc.parallel_loop(0, C // 16, 1)
        def gclear(q):
            dvec = idx_d[pl.ds(q * 16, 16)]
            colb = dvec & 127
            for l in range(16):
                e = q * 16 + l
                oh_buf[e, pl.ds((colb[l] >> 4) * 16, 16)] = zv

    stage_idx(0, idx_s0, idx_d0)
    issue(idx_s0, idx_d0, rows_s0, rows_d0, sem0)

    def pair_body(i, _):
        g = 2 * i
        stage_idx(g + 1, idx_s1, idx_d1)
        drain(idx_s0, idx_d0, rows_s0, rows_d0, sem0)
        issue(idx_s1, idx_d1, rows_s1, rows_d1, sem1)
        compute_scatter(idx_d0, rows_s0, rows_d0)
        stage_idx(g + 2, idx_s0, idx_d0)
        drain(idx_s1, idx_d1, rows_s1, rows_d1, sem1)
        issue(idx_s0, idx_d0, rows_s0, rows_d0, sem0)
        compute_scatter(idx_d1, rows_s1, rows_d1)
        return 0

    lax.fori_loop(0, NCHUNK // 2, pair_body, 0)
    drain(idx_s0, idx_d0, rows_s0, rows_d0, sem0)

    plsc.subcore_barrier()

    for b in range(ROWS_PER_TILE // ZROWS):
        r = row0 + b * ZROWS
        pltpu.sync_copy(acc.at[pl.ds(r, ZROWS)], out_hbm.at[c, pl.ds(r, ZROWS)])

    @pl.when(s == 0)
    def _():
        pltpu.sync_copy(acc.at[pl.ds(NP, NDEN)],
                        out_hbm.at[c, pl.ds(NP, NDEN)])


def _edge_pass(xl, xr, src, dst, att):
    mesh = plsc.VectorSubcoreMesh(core_axis_name="c", subcore_axis_name="s",
                                  num_cores=NC, num_subcores=NS)
    k = pl.kernel(
        _edge_kernel_body,
        out_type=jax.ShapeDtypeStruct((NC, NP_TOT, D), jnp.float32),
        mesh=mesh,
        scratch_types=[
            pltpu.VMEM_SHARED((NP_TOT, D), jnp.float32),
            pltpu.VMEM((C,), jnp.int32),
            pltpu.VMEM((C,), jnp.int32),
            pltpu.VMEM((C,), jnp.int32),
            pltpu.VMEM((C,), jnp.int32),
            pltpu.VMEM((C,), jnp.int32),
            pltpu.VMEM((C, D), jnp.float32),
            pltpu.VMEM((C, D), jnp.float32),
            pltpu.VMEM((C, D), jnp.float32),
            pltpu.VMEM((C, D), jnp.float32),
            pltpu.VMEM((C, D), jnp.float32),
            pltpu.VMEM((D,), jnp.float32),
            pltpu.SemaphoreType.DMA,
            pltpu.SemaphoreType.DMA,
        ],
    )
    return k(xl, xr, src, dst, att)


_BM = 256
_NBLK = NP // _BM


def _lin2_body(x_ref, wl_ref, wr_ref, xl_ref, xr_ref):
    xb = x_ref[...]
    xl_ref[...] = jnp.dot(xb, wl_ref[...], preferred_element_type=jnp.float32)
    xr_ref[...] = jnp.dot(xb, wr_ref[...], preferred_element_type=jnp.float32)


def _lin2(x, wl, wr):
    return pl.pallas_call(
        _lin2_body,
        grid=(_NBLK,),
        in_specs=[
            pl.BlockSpec((_BM, D), lambda i: (i, 0)),
            pl.BlockSpec((D, D), lambda i: (0, 0)),
            pl.BlockSpec((D, D), lambda i: (0, 0)),
        ],
        out_specs=[pl.BlockSpec((_BM, D), lambda i: (i, 0))] * 2,
        out_shape=[jax.ShapeDtypeStruct((NP, D), jnp.float32)] * 2,
    )(x, wl, wr)


def _combine_lin2_body(num_ref, den_ref, b_ref, wl_ref, wr_ref,
                       xl_ref, xr_ref):
    num = num_ref[0] + num_ref[1]
    den = den_ref[0] + den_ref[1]
    h = jnp.where(den > 0.0, num / den + b_ref[...], 0.0)
    xl_ref[...] = jnp.dot(h, wl_ref[...], preferred_element_type=jnp.float32)
    xr_ref[...] = jnp.dot(h, wr_ref[...], preferred_element_type=jnp.float32)


def _combine_lin2(num, den, b, wl, wr):
    return pl.pallas_call(
        _combine_lin2_body,
        grid=(_NBLK,),
        in_specs=[
            pl.BlockSpec((NC, _BM, D), lambda i: (0, i, 0)),
            pl.BlockSpec((NC, _BM, 1), lambda i: (0, i, 0)),
            pl.BlockSpec((1, D), lambda i: (0, 0)),
            pl.BlockSpec((D, D), lambda i: (0, 0)),
            pl.BlockSpec((D, D), lambda i: (0, 0)),
        ],
        out_specs=[pl.BlockSpec((_BM, D), lambda i: (i, 0))] * 2,
        out_shape=[jax.ShapeDtypeStruct((NP, D), jnp.float32)] * 2,
    )(num, den, b, wl, wr)


def _pool_body(num_ref, den_ref, b_ref, batch_ref, out_ref, sums, counts):
    i = pl.program_id(0)

    @pl.when(i == 0)
    def _():
        sums[...] = jnp.zeros_like(sums)
        counts[...] = jnp.zeros_like(counts)

    num = num_ref[0] + num_ref[1]
    den = den_ref[0] + den_ref[1]
    h = jnp.where(den > 0.0, num / den + b_ref[...], 0.0)
    gid = lax.broadcasted_iota(jnp.int32, (_BM, G), 1)
    onehot = (batch_ref[...] == gid).astype(jnp.float32)
    sums[...] += lax.dot_general(onehot, h, (((0,), (0,)), ((), ())),
                                 preferred_element_type=jnp.float32)
    counts[...] += jnp.sum(onehot, axis=0, keepdims=True)

    @pl.when(i == _NBLK - 1)
    def _():
        out_ref[...] = sums[...] / jnp.maximum(counts[...], 1.0).T


def _pool(num, den, b, batch2d):
    return pl.pallas_call(
        _pool_body,
        grid=(_NBLK,),
        in_specs=[
            pl.BlockSpec((NC, _BM, D), lambda i: (0, i, 0)),
            pl.BlockSpec((NC, _BM, 1), lambda i: (0, i, 0)),
            pl.BlockSpec((1, D), lambda i: (0, 0)),
            pl.BlockSpec((_BM, 1), lambda i: (i, 0)),
        ],
        out_specs=pl.BlockSpec((G, D), lambda i: (0, 0)),
        out_shape=jax.ShapeDtypeStruct((G, D), jnp.float32),
        scratch_shapes=[
            pltpu.VMEM((G, D), jnp.float32),
            pltpu.VMEM((1, G), jnp.float32),
        ],
    )(num, den, b, batch2d)


def _split_acc(acc):
    num = acc[:, :NP, :]
    den = acc[:, NP:, :].reshape(NC, NP, 1)
    return num, den


@jax.jit
def kernel(x, edge_index, batch, W_l1, W_r1, att1, b1, W_l2, W_r2, att2, b2):
    loops = jnp.arange(N, dtype=jnp.int32)
    npad = E_ALL - E_TOT
    src = jnp.concatenate([edge_index[0], loops,
                           jnp.zeros((npad,), jnp.int32)])
    dst = jnp.concatenate([edge_index[1], loops,
                           jnp.full((npad,), N, jnp.int32)])
    x_pad = jnp.pad(x, ((0, NP - N), (0, 0)))
    batch2d = jnp.pad(batch, (0, NP - N), constant_values=-1).reshape(NP, 1)

    xl1, xr1 = _lin2(x_pad, W_l1, W_r1)
    acc1 = _edge_pass(xl1, xr1, src, dst, att1)
    num1, den1 = _split_acc(acc1)
    xl2, xr2 = _combine_lin2(num1, den1, b1.reshape(1, D), W_l2, W_r2)
    acc2 = _edge_pass(xl2, xr2, src, dst, att2)
    num2, den2 = _split_acc(acc2)
    return _pool(num2, den2, b2.reshape(1, D), batch2d)


# speedup vs baseline: 1.2309x; 1.2309x over previous
import functools

import jax
import jax.numpy as jnp
from jax import lax
from jax.experimental import pallas as pl
from jax.experimental.pallas import tpu as pltpu
from jax.experimental.pallas import tpu_sc as plsc

N = 10000
D = 128
G = 8
NP = 10240
NDEN = NP // D
NP_TOT = NP + NDEN
NC, NS = 2, 16
C = 64
E_RAW = 320000
E_TOT = E_RAW + N
NCHUNK = -(-E_TOT // (NC * NS * C))
T = NCHUNK * C
E_PAD = NC * NS * T
E_ALL = E_PAD + C

ROWS_PER_TILE = NP // NS
ZROWS = C


def _lane_take(v, idx):
    return lax.gather(
        v, idx[:, None],
        dimension_numbers=lax.GatherDimensionNumbers(
            offset_dims=(), collapsed_slice_dims=(0,), start_index_map=(0,)),
        slice_sizes=(1,), mode=lax.GatherScatterMode.PROMISE_IN_BOUNDS)


def _lane_allsum(v, i16):
    for b in (8, 4, 2, 1):
        v = v + _lane_take(v, i16 ^ b)
    return v


def _edge_kernel_body(xl_hbm, xr_hbm, src_hbm, dst_hbm, att_hbm, out_hbm,
                      acc, idx_s0, idx_d0, idx_s1, idx_d1, idx_den,
                      rows_s0, rows_d0, rows_s1, rows_d1,
                      att_v, sem0, sem1):
    c = lax.axis_index("c")
    s = lax.axis_index("s")
    tid = c * NS + s

    zv = jnp.zeros((16,), jnp.float32)

    def zb_body(r, _):
        for j in range(D // 16):
            rows_s0[r, pl.ds(j * 16, 16)] = zv
        return 0
    lax.fori_loop(0, ZROWS, zb_body, 0)
    row0 = s * ROWS_PER_TILE
    for b in range(ROWS_PER_TILE // ZROWS):
        pltpu.sync_copy(rows_s0, acc.at[pl.ds(row0 + b * ZROWS, ZROWS)])

    @pl.when(s == 0)
    def _():
        pltpu.sync_copy(rows_s0, acc.at[pl.ds(NP, ZROWS)])
        pltpu.sync_copy(rows_s0.at[pl.ds(0, NDEN - ZROWS)],
                        acc.at[pl.ds(NP + ZROWS, NDEN - ZROWS)])

    pltpu.sync_copy(att_hbm, att_v)
    att_regs = [att_v[pl.ds(k * 16, 16)] for k in range(D // 16)]
    i16 = lax.iota(jnp.int32, 16)
    col_iotas = [i16 + k * 16 for k in range(D // 16)]

    plsc.subcore_barrier()

    ebase = tid * T

    def stage_idx(g, idx_s, idx_d):
        eb = ebase + g * C
        pltpu.sync_copy(src_hbm.at[pl.ds(eb, C)], idx_s)
        pltpu.sync_copy(dst_hbm.at[pl.ds(eb, C)], idx_d)

    def issue(idx_s, idx_d, rs, rd, sem):
        pltpu.async_copy(xl_hbm.at[idx_s], rs, sem)
        pltpu.async_copy(xr_hbm.at[idx_d], rd, sem)

    def drain(idx_s, idx_d, rs, rd, sem):
        pltpu.make_async_copy(xl_hbm.at[idx_s], rs, sem).wait()
        pltpu.make_async_copy(xr_hbm.at[idx_d], rd, sem).wait()

    def compute_scatter(idx_d, rows_s, rows_d):
        @plsc.parallel_loop(0, C // 16, 1)
        def gbody(q):
            dvec = idx_d[pl.ds(q * 16, 16)]
            idx_den[pl.ds(q * 16, 16)] = (dvec >> 7) + NP
            colb = dvec & 127
            for l in range(16):
                e = q * 16 + l
                acc0 = jnp.zeros((16,), jnp.float32)
                acc1 = jnp.zeros((16,), jnp.float32)
                for k in range(D // 16):
                    t = (rows_s[e, pl.ds(k * 16, 16)]
                         + rows_d[e, pl.ds(k * 16, 16)])
                    lr = jnp.maximum(t, 0.2 * t)
                    if k % 2 == 0:
                        acc0 = acc0 + lr * att_regs[k]
                    else:
                        acc1 = acc1 + lr * att_regs[k]
                wv = jnp.exp(_lane_allsum(acc0 + acc1, i16))
                colv = lax.broadcast(colb[l], (16,))
                for k in range(D // 16):
                    rows_s[e, pl.ds(k * 16, 16)] = (
                        rows_s[e, pl.ds(k * 16, 16)] * wv)
                    rows_d[e, pl.ds(k * 16, 16)] = jnp.where(
                        col_iotas[k] == colv, wv, 0.0)

        pltpu.sync_copy(rows_s, acc.at[idx_d], add=True)
        pltpu.sync_copy(rows_d, acc.at[idx_den], add=True)

    stage_idx(0, idx_s0, idx_d0)
    issue(idx_s0, idx_d0, rows_s0, rows_d0, sem0)

    def pair_body(i, _):
        g = 2 * i
        stage_idx(g + 1, idx_s1, idx_d1)
        drain(idx_s0, idx_d0, rows_s0, rows_d0, sem0)
        issue(idx_s1, idx_d1, rows_s1, rows_d1, sem1)
        compute_scatter(idx_d0, rows_s0, rows_d0)
        stage_idx(g + 2, idx_s0, idx_d0)
        drain(idx_s1, idx_d1, rows_s1, rows_d1, sem1)
        issue(idx_s0, idx_d0, rows_s0, rows_d0, sem0)
        compute_scatter(idx_d1, rows_s1, rows_d1)
        return 0

    lax.fori_loop(0, NCHUNK // 2, pair_body, 0)
    drain(idx_s0, idx_d0, rows_s0, rows_d0, sem0)

    plsc.subcore_barrier()

    for b in range(ROWS_PER_TILE // ZROWS):
        r = row0 + b * ZROWS
        pltpu.sync_copy(acc.at[pl.ds(r, ZROWS)], out_hbm.at[c, pl.ds(r, ZROWS)])

    @pl.when(s == 0)
    def _():
        pltpu.sync_copy(acc.at[pl.ds(NP, NDEN)],
                        out_hbm.at[c, pl.ds(NP, NDEN)])


def _edge_pass(xl, xr, src, dst, att):
    mesh = plsc.VectorSubcoreMesh(core_axis_name="c", subcore_axis_name="s",
                                  num_cores=NC, num_subcores=NS)
    k = pl.kernel(
        _edge_kernel_body,
        out_type=jax.ShapeDtypeStruct((NC, NP_TOT, D), jnp.float32),
        mesh=mesh,
        scratch_types=[
            pltpu.VMEM_SHARED((NP_TOT, D), jnp.float32),
            pltpu.VMEM((C,), jnp.int32),
            pltpu.VMEM((C,), jnp.int32),
            pltpu.VMEM((C,), jnp.int32),
            pltpu.VMEM((C,), jnp.int32),
            pltpu.VMEM((C,), jnp.int32),
            pltpu.VMEM((C, D), jnp.float32),
            pltpu.VMEM((C, D), jnp.float32),
            pltpu.VMEM((C, D), jnp.float32),
            pltpu.VMEM((C, D), jnp.float32),
            pltpu.VMEM((D,), jnp.float32),
            pltpu.SemaphoreType.DMA,
            pltpu.SemaphoreType.DMA,
        ],
    )
    return k(xl, xr, src, dst, att)


_BM = 256
_NBLK = NP // _BM


def _lin2_body(x_ref, wl_ref, wr_ref, xl_ref, xr_ref):
    xb = x_ref[...]
    xl_ref[...] = jnp.dot(xb, wl_ref[...], preferred_element_type=jnp.float32)
    xr_ref[...] = jnp.dot(xb, wr_ref[...], preferred_element_type=jnp.float32)


def _lin2(x, wl, wr):
    return pl.pallas_call(
        _lin2_body,
        grid=(_NBLK,),
        in_specs=[
            pl.BlockSpec((_BM, D), lambda i: (i, 0)),
            pl.BlockSpec((D, D), lambda i: (0, 0)),
            pl.BlockSpec((D, D), lambda i: (0, 0)),
        ],
        out_specs=[pl.BlockSpec((_BM, D), lambda i: (i, 0))] * 2,
        out_shape=[jax.ShapeDtypeStruct((NP, D), jnp.float32)] * 2,
    )(x, wl, wr)


def _combine_lin2_body(num_ref, den_ref, b_ref, wl_ref, wr_ref,
                       xl_ref, xr_ref):
    num = num_ref[0] + num_ref[1]
    den = den_ref[0] + den_ref[1]
    h = jnp.where(den > 0.0, num / den + b_ref[...], 0.0)
    xl_ref[...] = jnp.dot(h, wl_ref[...], preferred_element_type=jnp.float32)
    xr_ref[...] = jnp.dot(h, wr_ref[...], preferred_element_type=jnp.float32)


def _combine_lin2(num, den, b, wl, wr):
    return pl.pallas_call(
        _combine_lin2_body,
        grid=(_NBLK,),
        in_specs=[
            pl.BlockSpec((NC, _BM, D), lambda i: (0, i, 0)),
            pl.BlockSpec((NC, _BM, 1), lambda i: (0, i, 0)),
            pl.BlockSpec((1, D), lambda i: (0, 0)),
            pl.BlockSpec((D, D), lambda i: (0, 0)),
            pl.BlockSpec((D, D), lambda i: (0, 0)),
        ],
        out_specs=[pl.BlockSpec((_BM, D), lambda i: (i, 0))] * 2,
        out_shape=[jax.ShapeDtypeStruct((NP, D), jnp.float32)] * 2,
    )(num, den, b, wl, wr)


def _pool_body(num_ref, den_ref, b_ref, batch_ref, out_ref, sums, counts):
    i = pl.program_id(0)

    @pl.when(i == 0)
    def _():
        sums[...] = jnp.zeros_like(sums)
        counts[...] = jnp.zeros_like(counts)

    num = num_ref[0] + num_ref[1]
    den = den_ref[0] + den_ref[1]
    h = jnp.where(den > 0.0, num / den + b_ref[...], 0.0)
    gid = lax.broadcasted_iota(jnp.int32, (_BM, G), 1)
    onehot = (batch_ref[...] == gid).astype(jnp.float32)
    sums[...] += lax.dot_general(onehot, h, (((0,), (0,)), ((), ())),
                                 preferred_element_type=jnp.float32)
    counts[...] += jnp.sum(onehot, axis=0, keepdims=True)

    @pl.when(i == _NBLK - 1)
    def _():
        out_ref[...] = sums[...] / jnp.maximum(counts[...], 1.0).T


def _pool(num, den, b, batch2d):
    return pl.pallas_call(
        _pool_body,
        grid=(_NBLK,),
        in_specs=[
            pl.BlockSpec((NC, _BM, D), lambda i: (0, i, 0)),
            pl.BlockSpec((NC, _BM, 1), lambda i: (0, i, 0)),
            pl.BlockSpec((1, D), lambda i: (0, 0)),
            pl.BlockSpec((_BM, 1), lambda i: (i, 0)),
        ],
        out_specs=pl.BlockSpec((G, D), lambda i: (0, 0)),
        out_shape=jax.ShapeDtypeStruct((G, D), jnp.float32),
        scratch_shapes=[
            pltpu.VMEM((G, D), jnp.float32),
            pltpu.VMEM((1, G), jnp.float32),
        ],
    )(num, den, b, batch2d)


def _split_acc(acc):
    num = acc[:, :NP, :]
    den = acc[:, NP:, :].reshape(NC, NP, 1)
    return num, den


@jax.jit
def kernel(x, edge_index, batch, W_l1, W_r1, att1, b1, W_l2, W_r2, att2, b2):
    loops = jnp.arange(N, dtype=jnp.int32)
    npad = E_ALL - E_TOT
    src = jnp.concatenate([edge_index[0], loops,
                           jnp.zeros((npad,), jnp.int32)])
    dst = jnp.concatenate([edge_index[1], loops,
                           jnp.full((npad,), N, jnp.int32)])
    x_pad = jnp.pad(x, ((0, NP - N), (0, 0)))
    batch2d = jnp.pad(batch, (0, NP - N), constant_values=-1).reshape(NP, 1)

    xl1, xr1 = _lin2(x_pad, W_l1, W_r1)
    acc1 = _edge_pass(xl1, xr1, src, dst, att1)
    num1, den1 = _split_acc(acc1)
    xl2, xr2 = _combine_lin2(num1, den1, b1.reshape(1, D), W_l2, W_r2)
    acc2 = _edge_pass(xl2, xr2, src, dst, att2)
    num2, den2 = _split_acc(acc2)
    return _pool(num2, den2, b2.reshape(1, D), batch2d)


# speedup vs baseline: 1.7010x; 1.3819x over previous
import functools

import jax
import jax.numpy as jnp
from jax import lax
from jax.experimental import pallas as pl
from jax.experimental.pallas import tpu as pltpu
from jax.experimental.pallas import tpu_sc as plsc

N = 10000
D = 128
G = 8
NP = 10240
NDEN = NP // D
NP_TOT = NP + NDEN
NC, NS = 2, 16
C = 80
E_RAW = 320000
E_TOT = E_RAW + N
NCHUNK = -(-E_TOT // (NC * NS * C))
T = NCHUNK * C
E_PAD = NC * NS * T
E_ALL = E_PAD + C
NCH_ALL = E_ALL // C

ROWS_PER_TILE = NP // NS
ZROWS = C


def _lane_take(v, idx):
    return lax.gather(
        v, idx[:, None],
        dimension_numbers=lax.GatherDimensionNumbers(
            offset_dims=(), collapsed_slice_dims=(0,), start_index_map=(0,)),
        slice_sizes=(1,), mode=lax.GatherScatterMode.PROMISE_IN_BOUNDS)


def _lane_allsum(v, i16):
    for b in (8, 4, 2, 1):
        v = v + _lane_take(v, i16 ^ b)
    return v


def _edge_kernel_body(xl_hbm, xr_hbm, idx_hbm, att_hbm, out_hbm,
                      acc, idx0, idx1, sidx0, sidx1,
                      rows_s0, rows_d0, rows_s1, rows_d1,
                      att_v, sem0, sem1, ssem0, ssem1):
    c = lax.axis_index("c")
    s = lax.axis_index("s")
    tid = c * NS + s

    zv = jnp.zeros((16,), jnp.float32)

    def zb_body(r, _):
        for j in range(D // 16):
            rows_s0[r, pl.ds(j * 16, 16)] = zv
        return 0
    lax.fori_loop(0, ZROWS, zb_body, 0)
    row0 = s * ROWS_PER_TILE
    for b in range(ROWS_PER_TILE // ZROWS):
        pltpu.sync_copy(rows_s0, acc.at[pl.ds(row0 + b * ZROWS, ZROWS)])

    @pl.when(s == 0)
    def _():
        pltpu.sync_copy(rows_s0, acc.at[pl.ds(NP, NDEN)])

    pltpu.sync_copy(att_hbm, att_v)
    att_regs = [att_v[pl.ds(k * 16, 16)] for k in range(D // 16)]
    i16 = lax.iota(jnp.int32, 16)
    col_iotas = [i16 + k * 16 for k in range(D // 16)]

    plsc.subcore_barrier()

    cbase = tid * NCHUNK

    def stage_idx(g, idx):
        pltpu.sync_copy(idx_hbm.at[pl.ds((cbase + g) * 2, 2)], idx)

    def issue(idx, rs, rd, sem):
        pltpu.async_copy(xl_hbm.at[idx.at[0]], rs, sem)
        pltpu.async_copy(xr_hbm.at[idx.at[1]], rd, sem)

    def drain_gather(idx, rs, rd, sem):
        pltpu.make_async_copy(xl_hbm.at[idx.at[0]], rs, sem).wait()
        pltpu.make_async_copy(xr_hbm.at[idx.at[1]], rd, sem).wait()

    def compute(idx, sidx, rows_s, rows_d):
        @plsc.parallel_loop(0, C // 16, 1)
        def gbody(q):
            dvec = idx[1, pl.ds(q * 16, 16)]
            sidx[0, pl.ds(q * 16, 16)] = dvec
            sidx[1, pl.ds(q * 16, 16)] = (dvec >> 7) + NP
            colb = dvec & 127
            for l in range(16):
                e = q * 16 + l
                acc0 = jnp.zeros((16,), jnp.float32)
                acc1 = jnp.zeros((16,), jnp.float32)
                for k in range(D // 16):
                    t = (rows_s[e, pl.ds(k * 16, 16)]
                         + rows_d[e, pl.ds(k * 16, 16)])
                    lr = jnp.maximum(t, 0.2 * t)
                    if k % 2 == 0:
                        acc0 = acc0 + lr * att_regs[k]
                    else:
                        acc1 = acc1 + lr * att_regs[k]
                wv = jnp.exp(_lane_allsum(acc0 + acc1, i16))
                colv = lax.broadcast(colb[l], (16,))
                for k in range(D // 16):
                    rows_s[e, pl.ds(k * 16, 16)] = (
                        rows_s[e, pl.ds(k * 16, 16)] * wv)
                    rows_d[e, pl.ds(k * 16, 16)] = jnp.where(
                        col_iotas[k] == colv, wv, 0.0)

    def issue_scatter(sidx, rows_s, rows_d, ssem):
        pltpu.async_copy(rows_s, acc.at[sidx.at[0]], ssem, add=True)
        pltpu.async_copy(rows_d, acc.at[sidx.at[1]], ssem, add=True)

    def drain_scatter(sidx, rows_s, rows_d, ssem):
        pltpu.make_async_copy(rows_s, acc.at[sidx.at[0]], ssem).wait()
        pltpu.make_async_copy(rows_d, acc.at[sidx.at[1]], ssem).wait()

    stage_idx(0, idx0)
    issue(idx0, rows_s0, rows_d0, sem0)

    def pair_body(i, _):
        g = 2 * i
        stage_idx(g + 1, idx1)
        drain_gather(idx0, rows_s0, rows_d0, sem0)

        @pl.when(i > 0)
        def _():
            drain_scatter(sidx1, rows_s1, rows_d1, ssem1)
        issue(idx1, rows_s1, rows_d1, sem1)
        compute(idx0, sidx0, rows_s0, rows_d0)
        issue_scatter(sidx0, rows_s0, rows_d0, ssem0)
        stage_idx(g + 2, idx0)
        drain_gather(idx1, rows_s1, rows_d1, sem1)
        drain_scatter(sidx0, rows_s0, rows_d0, ssem0)
        issue(idx0, rows_s0, rows_d0, sem0)
        compute(idx1, sidx1, rows_s1, rows_d1)
        issue_scatter(sidx1, rows_s1, rows_d1, ssem1)
        return 0

    lax.fori_loop(0, NCHUNK // 2, pair_body, 0)
    drain_gather(idx0, rows_s0, rows_d0, sem0)
    drain_scatter(sidx1, rows_s1, rows_d1, ssem1)

    plsc.subcore_barrier()

    for b in range(ROWS_PER_TILE // ZROWS):
        r = row0 + b * ZROWS
        pltpu.sync_copy(acc.at[pl.ds(r, ZROWS)], out_hbm.at[c, pl.ds(r, ZROWS)])

    @pl.when(s == 0)
    def _():
        pltpu.sync_copy(acc.at[pl.ds(NP, NDEN)],
                        out_hbm.at[c, pl.ds(NP, NDEN)])


def _edge_pass(xl, xr, idx_packed, att):
    mesh = plsc.VectorSubcoreMesh(core_axis_name="c", subcore_axis_name="s",
                                  num_cores=NC, num_subcores=NS)
    k = pl.kernel(
        _edge_kernel_body,
        out_type=jax.ShapeDtypeStruct((NC, NP_TOT, D), jnp.float32),
        mesh=mesh,
        scratch_types=[
            pltpu.VMEM_SHARED((NP_TOT, D), jnp.float32),
            pltpu.VMEM((2, C), jnp.int32),
            pltpu.VMEM((2, C), jnp.int32),
            pltpu.VMEM((2, C), jnp.int32),
            pltpu.VMEM((2, C), jnp.int32),
            pltpu.VMEM((C, D), jnp.float32),
            pltpu.VMEM((C, D), jnp.float32),
            pltpu.VMEM((C, D), jnp.float32),
            pltpu.VMEM((C, D), jnp.float32),
            pltpu.VMEM((D,), jnp.float32),
            pltpu.SemaphoreType.DMA,
            pltpu.SemaphoreType.DMA,
            pltpu.SemaphoreType.DMA,
            pltpu.SemaphoreType.DMA,
        ],
    )
    return k(xl, xr, idx_packed, att)


_BM = 256
_NBLK = NP // _BM


def _lin2_body(x_ref, wl_ref, wr_ref, xl_ref, xr_ref):
    xb = x_ref[...]
    xl_ref[...] = jnp.dot(xb, wl_ref[...], preferred_element_type=jnp.float32)
    xr_ref[...] = jnp.dot(xb, wr_ref[...], preferred_element_type=jnp.float32)


def _lin2(x, wl, wr):
    return pl.pallas_call(
        _lin2_body,
        grid=(_NBLK,),
        in_specs=[
            pl.BlockSpec((_BM, D), lambda i: (i, 0)),
            pl.BlockSpec((D, D), lambda i: (0, 0)),
            pl.BlockSpec((D, D), lambda i: (0, 0)),
        ],
        out_specs=[pl.BlockSpec((_BM, D), lambda i: (i, 0))] * 2,
        out_shape=[jax.ShapeDtypeStruct((NP, D), jnp.float32)] * 2,
    )(x, wl, wr)


def _combine_lin2_body(num_ref, den_ref, b_ref, wl_ref, wr_ref,
                       xl_ref, xr_ref):
    num = num_ref[0] + num_ref[1]
    den = den_ref[0] + den_ref[1]
    h = jnp.where(den > 0.0, num / den + b_ref[...], 0.0)
    xl_ref[...] = jnp.dot(h, wl_ref[...], preferred_element_type=jnp.float32)
    xr_ref[...] = jnp.dot(h, wr_ref[...], preferred_element_type=jnp.float32)


def _combine_lin2(num, den, b, wl, wr):
    return pl.pallas_call(
        _combine_lin2_body,
        grid=(_NBLK,),
        in_specs=[
            pl.BlockSpec((NC, _BM, D), lambda i: (0, i, 0)),
            pl.BlockSpec((NC, _BM, 1), lambda i: (0, i, 0)),
            pl.BlockSpec((1, D), lambda i: (0, 0)),
            pl.BlockSpec((D, D), lambda i: (0, 0)),
            pl.BlockSpec((D, D), lambda i: (0, 0)),
        ],
        out_specs=[pl.BlockSpec((_BM, D), lambda i: (i, 0))] * 2,
        out_shape=[jax.ShapeDtypeStruct((NP, D), jnp.float32)] * 2,
    )(num, den, b, wl, wr)


def _pool_body(num_ref, den_ref, b_ref, batch_ref, out_ref, sums, counts):
    i = pl.program_id(0)

    @pl.when(i == 0)
    def _():
        sums[...] = jnp.zeros_like(sums)
        counts[...] = jnp.zeros_like(counts)

    num = num_ref[0] + num_ref[1]
    den = den_ref[0] + den_ref[1]
    h = jnp.where(den > 0.0, num / den + b_ref[...], 0.0)
    gid = lax.broadcasted_iota(jnp.int32, (_BM, G), 1)
    onehot = (batch_ref[...] == gid).astype(jnp.float32)
    sums[...] += lax.dot_general(onehot, h, (((0,), (0,)), ((), ())),
                                 preferred_element_type=jnp.float32)
    counts[...] += jnp.sum(onehot, axis=0, keepdims=True)

    @pl.when(i == _NBLK - 1)
    def _():
        out_ref[...] = sums[...] / jnp.maximum(counts[...], 1.0).T


def _pool(num, den, b, batch2d):
    return pl.pallas_call(
        _pool_body,
        grid=(_NBLK,),
        in_specs=[
            pl.BlockSpec((NC, _BM, D), lambda i: (0, i, 0)),
            pl.BlockSpec((NC, _BM, 1), lambda i: (0, i, 0)),
            pl.BlockSpec((1, D), lambda i: (0, 0)),
            pl.BlockSpec((_BM, 1), lambda i: (i, 0)),
        ],
        out_specs=pl.BlockSpec((G, D), lambda i: (0, 0)),
        out_shape=jax.ShapeDtypeStruct((G, D), jnp.float32),
        scratch_shapes=[
            pltpu.VMEM((G, D), jnp.float32),
            pltpu.VMEM((1, G), jnp.float32),
        ],
    )(num, den, b, batch2d)


def _split_acc(acc):
    num = acc[:, :NP, :]
    den = acc[:, NP:, :].reshape(NC, NP, 1)
    return num, den


@jax.jit
def kernel(x, edge_index, batch, W_l1, W_r1, att1, b1, W_l2, W_r2, att2, b2):
    loops = jnp.arange(N, dtype=jnp.int32)
    npad = E_ALL - E_TOT
    src = jnp.concatenate([edge_index[0], loops,
                           jnp.zeros((npad,), jnp.int32)])
    dst = jnp.concatenate([edge_index[1], loops,
                           jnp.full((npad,), N, jnp.int32)])
    idx_packed = jnp.stack(
        [src.reshape(NCH_ALL, C), dst.reshape(NCH_ALL, C)],
        axis=1).reshape(2 * NCH_ALL, C)
    x_pad = jnp.pad(x, ((0, NP - N), (0, 0)))
    batch2d = jnp.pad(batch, (0, NP - N), constant_values=-1).reshape(NP, 1)

    xl1, xr1 = _lin2(x_pad, W_l1, W_r1)
    acc1 = _edge_pass(xl1, xr1, idx_packed, att1)
    num1, den1 = _split_acc(acc1)
    xl2, xr2 = _combine_lin2(num1, den1, b1.reshape(1, D), W_l2, W_r2)
    acc2 = _edge_pass(xl2, xr2, idx_packed, att2)
    num2, den2 = _split_acc(acc2)
    return _pool(num2, den2, b2.reshape(1, D), batch2d)
